# baseline (speedup 1.0000x reference)
import functools

import jax
import jax.numpy as jnp
from jax.experimental import pallas as pl
from jax.experimental.pallas import tpu as pltpu

_NBUF = 3
_BM = 400


def _gcn_body(a_hbm, x_hbm, w_ref, o_hbm, a_buf, x_buf, o_buf,
              a_sems, x_sem, o_sems, *, nsteps):
    def a_copy(i):
        slot = i % _NBUF
        return pltpu.make_async_copy(
            a_hbm.at[pl.ds(i * _BM, _BM), :], a_buf.at[slot], a_sems.at[slot]
        )

    def o_copy(i):
        slot = i % 2
        return pltpu.make_async_copy(
            o_buf.at[slot], o_hbm.at[pl.ds(i * _BM, _BM), :], o_sems.at[slot]
        )

    for j in range(_NBUF):
        a_copy(j).start()
    x_copy = pltpu.make_async_copy(x_hbm, x_buf, x_sem)
    x_copy.start()
    x_copy.wait()

    w_row = w_ref[...]
    for i in range(nsteps):
        a_copy(i).wait()
        if i >= 2:
            o_copy(i - 2).wait()
        acc = jnp.dot(
            a_buf[i % _NBUF], x_buf[...], preferred_element_type=jnp.float32
        )
        o_buf[i % 2] = acc * w_row
        o_copy(i).start()
        if i + _NBUF < nsteps:
            a_copy(i + _NBUF).start()
    o_copy(nsteps - 2).wait()
    o_copy(nsteps - 1).wait()


def kernel(input, A, W):
    n, d = input.shape
    nsteps = n // _BM
    return pl.pallas_call(
        functools.partial(_gcn_body, nsteps=nsteps),
        in_specs=[
            pl.BlockSpec(memory_space=pltpu.MemorySpace.HBM),
            pl.BlockSpec(memory_space=pltpu.MemorySpace.HBM),
            pl.BlockSpec(memory_space=pltpu.MemorySpace.VMEM),
        ],
        out_specs=pl.BlockSpec(memory_space=pltpu.MemorySpace.HBM),
        out_shape=jax.ShapeDtypeStruct((n, d), jnp.float32),
        scratch_shapes=[
            pltpu.VMEM((_NBUF, _BM, n), jnp.float32),
            pltpu.VMEM((n, d), jnp.float32),
            pltpu.VMEM((2, _BM, d), jnp.float32),
            pltpu.SemaphoreType.DMA((_NBUF,)),
            pltpu.SemaphoreType.DMA,
            pltpu.SemaphoreType.DMA((2,)),
        ],
    )(A, input, W)


# speedup vs baseline: 1.0706x; 1.0706x over previous
import jax
import jax.numpy as jnp
from jax.experimental import pallas as pl
from jax.experimental.pallas import tpu as pltpu


def _gcn_body(a_ref, x_ref, w_ref, o_ref):
    acc = jnp.dot(a_ref[...], x_ref[...], preferred_element_type=jnp.float32)
    o_ref[...] = acc * w_ref[...]


def kernel(input, A, W):
    n, d = input.shape
    bm = 400
    return pl.pallas_call(
        _gcn_body,
        grid=(n // bm,),
        in_specs=[
            pl.BlockSpec((bm, n), lambda m: (m, 0)),
            pl.BlockSpec((n, d), lambda m: (0, 0)),
            pl.BlockSpec((d,), lambda m: (0,)),
        ],
        out_specs=pl.BlockSpec((bm, d), lambda m: (m, 0)),
        out_shape=jax.ShapeDtypeStruct((n, d), jnp.float32),
        compiler_params=pltpu.CompilerParams(
            dimension_semantics=("arbitrary",),
        ),
    )(A, input, W)


# speedup vs baseline: 1.0712x; 1.0006x over previous
import jax
import jax.numpy as jnp
from jax.experimental import pallas as pl
from jax.experimental.pallas import tpu as pltpu


def _gcn_body(a_ref, x_ref, w_ref, o_ref, xb_ref):
    @pl.when(pl.program_id(0) == 0)
    def _cvt():
        xb_ref[...] = x_ref[...].astype(jnp.bfloat16)

    acc = jnp.dot(a_ref[...], xb_ref[...], preferred_element_type=jnp.float32)
    o_ref[...] = acc * w_ref[...]


def kernel(input, A, W):
    n, d = input.shape
    bm = 400
    return pl.pallas_call(
        _gcn_body,
        grid=(n // bm,),
        in_specs=[
            pl.BlockSpec((bm, n), lambda m: (m, 0)),
            pl.BlockSpec((n, d), lambda m: (0, 0)),
            pl.BlockSpec((d,), lambda m: (0,)),
        ],
        out_specs=pl.BlockSpec((bm, d), lambda m: (m, 0)),
        out_shape=jax.ShapeDtypeStruct((n, d), jnp.float32),
        scratch_shapes=[pltpu.VMEM((n, d), jnp.bfloat16)],
        compiler_params=pltpu.CompilerParams(
            dimension_semantics=("arbitrary",),
        ),
    )(A, input, W)
